# baseline (speedup 1.0000x reference)
import functools

import jax
import jax.numpy as jnp
from jax import lax
from jax.experimental import pallas as pl
from jax.experimental.pallas import tpu as pltpu
from jax.experimental.pallas import tpu_sc as plsc

NC, NS, L = 2, 16, 16
NW = NC * NS

N = 100000
D = 128
B = 2048
H = 256
EPS = 1e-5

R_BLK = 128
NBUF = 4
PF = 2
CH = ((N + NW - 1) // NW + 7) // 8 * 8
NPAD = NW * CH
ACC_R = B + 8 * NS
ZPT = ACC_R // NS


def _sc_body(feats, seg, sm, wawn, bawn, zrows, wout, part, acc,
             fb0, fb1, fb2, fb3, s20, s21, s22, s23,
             segb, smb, wcb, wvb, bab, sem_in, sem_sc, sem_m):
    fb = [fb0, fb1, fb2, fb3]
    s2b = [s20, s21, s22, s23]

    c = lax.axis_index("c")
    s = lax.axis_index("s")
    wid = s * NC + c

    start = wid * CH
    chunk = jnp.minimum(CH, N - start)
    nblk = (chunk + R_BLK - 1) // R_BLK

    def off_of(j):
        off0 = start + j * R_BLK
        return off0, jnp.minimum(off0, start + chunk - R_BLK)

    def issue_in(j, b):
        _, off = off_of(j)
        pltpu.async_copy(feats.at[pl.ds(off, R_BLK)], fb[b], sem_in.at[b])

    def wait_in(j, b):
        _, off = off_of(j)
        pltpu.make_async_copy(feats.at[pl.ds(off, R_BLK)], fb[b],
                              sem_in.at[b]).wait()

    for j0 in range(PF):
        issue_in(j0, j0)
    pltpu.async_copy(seg.at[pl.ds(start, CH)], segb, sem_m.at[0])
    pltpu.async_copy(sm.at[pl.ds(start, CH)], smb, sem_m.at[0])
    pltpu.async_copy(wawn, wvb, sem_m.at[0])
    pltpu.async_copy(bawn, bab, sem_m.at[0])

    pltpu.sync_copy(zrows.at[pl.ds(s * ZPT, ZPT)], acc.at[pl.ds(s * ZPT, ZPT)])

    pltpu.make_async_copy(seg.at[pl.ds(start, CH)], segb, sem_m.at[0]).wait()
    pltpu.make_async_copy(sm.at[pl.ds(start, CH)], smb, sem_m.at[0]).wait()
    pltpu.make_async_copy(wawn, wvb, sem_m.at[0]).wait()
    pltpu.make_async_copy(bawn, bab, sem_m.at[0]).wait()
    plsc.subcore_barrier()

    wv = [wvb[pl.ds(k * L, L)] for k in range(D // L)]
    bvn16 = bab[...]
    lane = lax.iota(jnp.int32, L)
    ohf = [jnp.where(lane == r, 1.0, 0.0).astype(jnp.float32) for r in range(L)]
    bfly = [lane ^ sh for sh in (8, 4, 2, 1)]

    def compute(j, b):
        off0, off = off_of(j)
        rel0 = off - start

        def grp_body(g, carry2):
            gbase = g * L
            rel = rel0 + gbase
            gidx = off + gbase + lane
            segv = segb[pl.ds(rel, L)]
            seg2 = jnp.where(gidx >= off0, segv, B)
            s2b[b][pl.ds(gbase, L)] = seg2
            smv = smb[pl.ds(rel, L)]
            wacc = jnp.zeros((L,), jnp.float32)
            for r in range(L):
                row = gbase + r
                chunks = [fb[b][row, pl.ds(kk * L, L)] for kk in range(D // L)]
                accv = bvn16 + chunks[0] * wv[0]
                for kk in range(1, D // L):
                    accv = accv + chunks[kk] * wv[kk]
                for idx in bfly:
                    accv = accv + jnp.take(accv, idx)
                sig = 1.0 / (1.0 + jnp.exp(accv))
                wrow = sig * smv[r]
                wacc = wacc + wrow * ohf[r]
                for kk in range(D // L):
                    fb[b][row, pl.ds(kk * L, L)] = chunks[kk] * wrow
            wcb[pl.ds(rel, L)] = wacc
            return carry2

        lax.fori_loop(0, R_BLK // L, grp_body, 0)

    def phase(j, b):
        wait_in(j, b)
        compute(j, b)
        pltpu.async_copy(fb[b], acc.at[s2b[b]], sem_sc.at[b], add=True)
        bp_ = (b + PF) % NBUF

        @pl.when(j + PF < nblk)
        def _():
            @pl.when(j >= NBUF - PF)
            def _():
                pltpu.make_async_copy(fb[bp_], acc.at[s2b[bp_]],
                                      sem_sc.at[bp_]).wait()
            issue_in(j + PF, bp_)

    nmac = (nblk + NBUF - 1) // NBUF

    def mac_body(m, carry):
        for p in range(NBUF):
            j = m * NBUF + p

            @pl.when(j < nblk)
            def _():
                phase(j, p)
        return carry

    lax.fori_loop(0, nmac, mac_body, 0)

    for b in range(NBUF):
        pltpu.make_async_copy(fb[b], acc.at[s2b[b]], sem_sc.at[b]).wait()

    pltpu.sync_copy(wcb, wout.at[pl.ds(start, CH)])

    plsc.subcore_barrier()
    rpt = B // NS
    pltpu.sync_copy(acc.at[pl.ds(s * rpt, rpt)], part.at[c, pl.ds(s * rpt, rpt)])


_sc_call = pl.kernel(
    _sc_body,
    out_type=(
        jax.ShapeDtypeStruct((NPAD,), jnp.float32),
        jax.ShapeDtypeStruct((NC, B, D), jnp.float32),
    ),
    mesh=plsc.VectorSubcoreMesh(
        core_axis_name="c", subcore_axis_name="s", num_cores=NC, num_subcores=NS
    ),
    scratch_types=(
        [pltpu.VMEM_SHARED((ACC_R, D), jnp.float32)]
        + [pltpu.VMEM((R_BLK, D), jnp.float32)] * NBUF
        + [pltpu.VMEM((R_BLK,), jnp.int32)] * NBUF
        + [pltpu.VMEM((CH,), jnp.int32),
           pltpu.VMEM((CH,), jnp.float32),
           pltpu.VMEM((CH,), jnp.float32),
           pltpu.VMEM((D,), jnp.float32),
           pltpu.VMEM((L,), jnp.float32),
           pltpu.SemaphoreType.DMA((NBUF,)),
           pltpu.SemaphoreType.DMA((NBUF,)),
           pltpu.SemaphoreType.DMA((1,))]
    ),
)


def _ffn_body(part, W1, b1, g1, be1, W2, b2, g2, be2, W3, b3, g3, be3,
              Wp, bp, out):
    h = part[0] + part[1]

    def fc(x, W, b, g, be):
        y = lax.dot_general(x, W[...], (((1,), (1,)), ((), ())),
                            preferred_element_type=jnp.float32) + b[...]
        y = jnp.maximum(y, 0.0)
        m = jnp.mean(y, axis=0, keepdims=True)
        v = jnp.mean(y * y, axis=0, keepdims=True) - m * m
        return (y - m) * lax.rsqrt(v + EPS) * g[...] + be[...]

    h = fc(h, W1, b1, g1, be1)
    h = fc(h, W2, b2, g2, be2)
    h = fc(h, W3, b3, g3, be3)
    out[...] = jnp.sum(h * Wp[...], axis=1, keepdims=True) + bp[...]


def _ffn_call(part, W1, b1, g1, be1, W2, b2, g2, be2, W3, b3, g3, be3, Wp, bp):
    return pl.pallas_call(
        _ffn_body,
        out_shape=jax.ShapeDtypeStruct((B, 1), jnp.float32),
    )(part, W1, b1, g1, be1, W2, b2, g2, be2, W3, b3, g3, be3, Wp, bp)


def kernel(segment_ids, rgcn_node_feats, rgcn_edge_feats, smask_feats,
           W_aw, b_aw, W1, b1, g1, be1, W2, b2, g2, be2, W3, b3, g3, be3,
           Wp, bp):
    seg = jnp.pad(segment_ids.astype(jnp.int32), (0, NPAD - N))
    sm = jnp.pad(smask_feats.reshape(N).astype(jnp.float32), (0, NPAD - N))
    wawn = (-W_aw).reshape(D).astype(jnp.float32)
    bawn = jnp.broadcast_to((-b_aw / L).astype(jnp.float32).reshape(1), (L,))
    zrows = jnp.zeros((ACC_R, D), jnp.float32)

    wout, part = _sc_call(rgcn_node_feats, seg, sm, wawn, bawn, zrows)

    out = _ffn_call(part,
                    W1, b1.reshape(1, H), g1.reshape(1, H), be1.reshape(1, H),
                    W2, b2.reshape(1, H), g2.reshape(1, H), be2.reshape(1, H),
                    W3, b3.reshape(1, H), g3.reshape(1, H), be3.reshape(1, H),
                    Wp, bp.reshape(1, 1))
    return (out, wout[:N].reshape(N, 1))


# speedup vs baseline: 1.0738x; 1.0738x over previous
import functools

import jax
import jax.numpy as jnp
from jax import lax
from jax.experimental import pallas as pl
from jax.experimental.pallas import tpu as pltpu
from jax.experimental.pallas import tpu_sc as plsc

NC, NS, L = 2, 16, 16
NW = NC * NS

N = 100000
D = 128
B = 2048
H = 256
EPS = 1e-5

R_BLK = 128
NBUF = 4
CH = ((N + NW - 1) // NW + 7) // 8 * 8
ACC_R = B + 8 * NS
ZPT = ACC_R // NS


def _sc_body(feats, seg, sm, wawn, bawn, wout, part, acc,
             fb0, fb1, fb2, fb3, sg0, sg1, sg2, sg3, s20, s21, s22, s23,
             sm0, sm1, sm2, sm3, wo0, wo1, wo2, wo3, wvb, bab,
             sem_in, sem_sc, sem_w):
    fb = [fb0, fb1, fb2, fb3]
    sgb = [sg0, sg1, sg2, sg3]
    s2b = [s20, s21, s22, s23]
    smb = [sm0, sm1, sm2, sm3]
    wob = [wo0, wo1, wo2, wo3]

    c = lax.axis_index("c")
    s = lax.axis_index("s")
    wid = s * NC + c

    start = wid * CH
    chunk = jnp.minimum(CH, N - start)
    nblk = (chunk + R_BLK - 1) // R_BLK

    def off_of(j):
        off0 = start + j * R_BLK
        return off0, jnp.minimum(off0, start + chunk - R_BLK)

    def issue_in(j, b):
        _, off = off_of(j)
        pltpu.async_copy(feats.at[pl.ds(off, R_BLK)], fb[b], sem_in.at[b])
        pltpu.async_copy(seg.at[pl.ds(off, R_BLK)], sgb[b], sem_in.at[b])
        pltpu.async_copy(sm.at[pl.ds(off, R_BLK)], smb[b], sem_in.at[b])

    def wait_in(j, b):
        _, off = off_of(j)
        pltpu.make_async_copy(feats.at[pl.ds(off, R_BLK)], fb[b], sem_in.at[b]).wait()
        pltpu.make_async_copy(seg.at[pl.ds(off, R_BLK)], sgb[b], sem_in.at[b]).wait()
        pltpu.make_async_copy(sm.at[pl.ds(off, R_BLK)], smb[b], sem_in.at[b]).wait()

    issue_in(0, 0)
    issue_in(1, 1)
    pltpu.async_copy(wawn, wvb, sem_w.at[0])
    pltpu.async_copy(bawn, bab, sem_w.at[0])

    def zbody(i, carry):
        for kk in range(D // L):
            fb3[i, pl.ds(kk * L, L)] = jnp.zeros((L,), jnp.float32)
        return carry

    lax.fori_loop(0, R_BLK, zbody, 0)
    pltpu.sync_copy(fb3, acc.at[pl.ds(s * ZPT, R_BLK)])
    pltpu.sync_copy(fb3.at[pl.ds(0, ZPT - R_BLK)],
                    acc.at[pl.ds(s * ZPT + R_BLK, ZPT - R_BLK)])
    pltpu.make_async_copy(wawn, wvb, sem_w.at[0]).wait()
    pltpu.make_async_copy(bawn, bab, sem_w.at[0]).wait()
    plsc.subcore_barrier()

    wv = [wvb[pl.ds(k * L, L)] for k in range(D // L)]
    bvn16 = bab[...]
    lane = lax.iota(jnp.int32, L)
    ohf = [jnp.where(lane == r, 1.0, 0.0).astype(jnp.float32) for r in range(L)]
    bfly = [lane ^ sh for sh in (8, 4, 2, 1)]

    def compute(j, b):
        off0, off = off_of(j)

        def grp_body(g, carry2):
            gbase = g * L
            gidx = off + gbase + lane
            segv = sgb[b][pl.ds(gbase, L)]
            seg2 = jnp.where(gidx >= off0, segv, B)
            s2b[b][pl.ds(gbase, L)] = seg2
            smv = smb[b][pl.ds(gbase, L)]
            wacc = jnp.zeros((L,), jnp.float32)
            for r in range(L):
                row = gbase + r
                chunks = [fb[b][row, pl.ds(kk * L, L)] for kk in range(D // L)]
                accv = bvn16 + chunks[0] * wv[0]
                for kk in range(1, D // L):
                    accv = accv + chunks[kk] * wv[kk]
                for idx in bfly:
                    accv = accv + jnp.take(accv, idx)
                sig = 1.0 / (1.0 + jnp.exp(accv))
                wrow = sig * smv[r]
                wacc = wacc + wrow * ohf[r]
                for kk in range(D // L):
                    fb[b][row, pl.ds(kk * L, L)] = chunks[kk] * wrow
            wob[b][pl.ds(gbase, L)] = wacc
            return carry2

        lax.fori_loop(0, R_BLK // L, grp_body, 0)

    def phase(j, b):
        @pl.when(j >= NBUF)
        def _():
            pltpu.make_async_copy(wob[b], wout.at[pl.ds(0, R_BLK)],
                                  sem_w.at[b]).wait()
        wait_in(j, b)
        compute(j, b)
        _, off = off_of(j)
        pltpu.async_copy(fb[b], acc.at[s2b[b]], sem_sc.at[b], add=True)
        pltpu.async_copy(wob[b], wout.at[pl.ds(off, R_BLK)], sem_w.at[b])
        b2 = (b + 2) % NBUF

        @pl.when(j + 2 < nblk)
        def _():
            @pl.when(j >= 2)
            def _():
                pltpu.make_async_copy(fb[b2], acc.at[s2b[b2]],
                                      sem_sc.at[b2]).wait()
            issue_in(j + 2, b2)

    nmac = (nblk + NBUF - 1) // NBUF

    def mac_body(m, carry):
        for p in range(NBUF):
            j = m * NBUF + p

            @pl.when(j < nblk)
            def _():
                phase(j, p)
        return carry

    lax.fori_loop(0, nmac, mac_body, 0)

    for b in range(NBUF):
        pltpu.make_async_copy(fb[b], acc.at[s2b[b]], sem_sc.at[b]).wait()
        pltpu.make_async_copy(wob[b], wout.at[pl.ds(0, R_BLK)],
                              sem_w.at[b]).wait()

    plsc.subcore_barrier()
    rpt = B // NS
    pltpu.sync_copy(acc.at[pl.ds(s * rpt, rpt)], part.at[c, pl.ds(s * rpt, rpt)])


_sc_call = pl.kernel(
    _sc_body,
    out_type=(
        jax.ShapeDtypeStruct((N,), jnp.float32),
        jax.ShapeDtypeStruct((NC, B, D), jnp.float32),
    ),
    mesh=plsc.VectorSubcoreMesh(
        core_axis_name="c", subcore_axis_name="s", num_cores=NC, num_subcores=NS
    ),
    scratch_types=(
        [pltpu.VMEM_SHARED((ACC_R, D), jnp.float32)]
        + [pltpu.VMEM((R_BLK, D), jnp.float32)] * NBUF
        + [pltpu.VMEM((R_BLK,), jnp.int32)] * NBUF
        + [pltpu.VMEM((R_BLK,), jnp.int32)] * NBUF
        + [pltpu.VMEM((R_BLK,), jnp.float32)] * NBUF
        + [pltpu.VMEM((R_BLK,), jnp.float32)] * NBUF
        + [pltpu.VMEM((D,), jnp.float32),
           pltpu.VMEM((L,), jnp.float32),
           pltpu.SemaphoreType.DMA((NBUF,)),
           pltpu.SemaphoreType.DMA((NBUF,)),
           pltpu.SemaphoreType.DMA((NBUF,))]
    ),
)


def _ffn_body(part, W1, b1, g1, be1, W2, b2, g2, be2, W3, b3, g3, be3,
              Wp, bp, out):
    h = part[0] + part[1]

    def fc(x, W, b, g, be):
        y = lax.dot_general(x, W[...], (((1,), (1,)), ((), ())),
                            preferred_element_type=jnp.float32) + b[...]
        y = jnp.maximum(y, 0.0)
        m = jnp.mean(y, axis=0, keepdims=True)
        v = jnp.mean(y * y, axis=0, keepdims=True) - m * m
        return (y - m) * lax.rsqrt(v + EPS) * g[...] + be[...]

    h = fc(h, W1, b1, g1, be1)
    h = fc(h, W2, b2, g2, be2)
    h = fc(h, W3, b3, g3, be3)
    out[...] = jnp.sum(h * Wp[...], axis=1, keepdims=True) + bp[...]


def _ffn_call(part, W1, b1, g1, be1, W2, b2, g2, be2, W3, b3, g3, be3, Wp, bp):
    return pl.pallas_call(
        _ffn_body,
        out_shape=jax.ShapeDtypeStruct((B, 1), jnp.float32),
    )(part, W1, b1, g1, be1, W2, b2, g2, be2, W3, b3, g3, be3, Wp, bp)


def kernel(segment_ids, rgcn_node_feats, rgcn_edge_feats, smask_feats,
           W_aw, b_aw, W1, b1, g1, be1, W2, b2, g2, be2, W3, b3, g3, be3,
           Wp, bp):
    seg = segment_ids.astype(jnp.int32)
    sm = smask_feats.reshape(N).astype(jnp.float32)
    wawn = (-W_aw).reshape(D).astype(jnp.float32)
    bawn = jnp.broadcast_to((-b_aw / L).astype(jnp.float32).reshape(1), (L,))

    wout, part = _sc_call(rgcn_node_feats, seg, sm, wawn, bawn)

    out = _ffn_call(part,
                    W1, b1.reshape(1, H), g1.reshape(1, H), be1.reshape(1, H),
                    W2, b2.reshape(1, H), g2.reshape(1, H), be2.reshape(1, H),
                    W3, b3.reshape(1, H), g3.reshape(1, H), be3.reshape(1, H),
                    Wp, bp.reshape(1, 1))
    return (out, wout.reshape(N, 1))


# speedup vs baseline: 1.0817x; 1.0073x over previous
import functools

import jax
import jax.numpy as jnp
from jax import lax
from jax.experimental import pallas as pl
from jax.experimental.pallas import tpu as pltpu
from jax.experimental.pallas import tpu_sc as plsc

NC, NS, L = 2, 16, 16
NW = NC * NS

N = 100000
D = 128
B = 2048
H = 256
EPS = 1e-5

R_BLK = 128
NBUF = 4
CH = ((N + NW - 1) // NW + 7) // 8 * 8
ACC_R = B + 8 * NS
ZPT = ACC_R // NS


def _sc_body(feats, seg, sm, wawn, bawn, wout, part, acc,
             fb0, fb1, fb2, fb3, sg0, sg1, sg2, sg3, s20, s21, s22, s23,
             sm0, sm1, sm2, sm3, wo0, wo1, wo2, wo3, wvb, bab,
             sem_in, sem_sc, sem_w):
    fb = [fb0, fb1, fb2, fb3]
    sgb = [sg0, sg1, sg2, sg3]
    s2b = [s20, s21, s22, s23]
    smb = [sm0, sm1, sm2, sm3]
    wob = [wo0, wo1, wo2, wo3]

    c = lax.axis_index("c")
    s = lax.axis_index("s")
    wid = s * NC + c

    start = wid * CH
    chunk = jnp.minimum(CH, N - start)
    nblk = (chunk + R_BLK - 1) // R_BLK

    def off_of(j):
        off0 = start + j * R_BLK
        return off0, jnp.minimum(off0, start + chunk - R_BLK)

    def issue_in(j, b):
        _, off = off_of(j)
        pltpu.async_copy(feats.at[pl.ds(off, R_BLK)], fb[b], sem_in.at[b])
        pltpu.async_copy(seg.at[pl.ds(off, R_BLK)], sgb[b], sem_in.at[b])
        pltpu.async_copy(sm.at[pl.ds(off, R_BLK)], smb[b], sem_in.at[b])

    def wait_in(j, b):
        _, off = off_of(j)
        pltpu.make_async_copy(feats.at[pl.ds(off, R_BLK)], fb[b], sem_in.at[b]).wait()
        pltpu.make_async_copy(seg.at[pl.ds(off, R_BLK)], sgb[b], sem_in.at[b]).wait()
        pltpu.make_async_copy(sm.at[pl.ds(off, R_BLK)], smb[b], sem_in.at[b]).wait()

    issue_in(0, 0)
    issue_in(1, 1)
    pltpu.async_copy(wawn, wvb, sem_w.at[0])
    pltpu.async_copy(bawn, bab, sem_w.at[0])

    def zbody(i, carry):
        for kk in range(D // L):
            fb3[i, pl.ds(kk * L, L)] = jnp.zeros((L,), jnp.float32)
        return carry

    lax.fori_loop(0, R_BLK, zbody, 0)
    pltpu.sync_copy(fb3, acc.at[pl.ds(s * ZPT, R_BLK)])
    pltpu.sync_copy(fb3.at[pl.ds(0, ZPT - R_BLK)],
                    acc.at[pl.ds(s * ZPT + R_BLK, ZPT - R_BLK)])
    pltpu.make_async_copy(wawn, wvb, sem_w.at[0]).wait()
    pltpu.make_async_copy(bawn, bab, sem_w.at[0]).wait()
    plsc.subcore_barrier()

    wv = [wvb[pl.ds(k * L, L)] for k in range(D // L)]
    bvn16 = bab[...]
    lane = lax.iota(jnp.int32, L)
    ohf = [jnp.where(lane == r, 1.0, 0.0).astype(jnp.float32) for r in range(L)]
    bfly = [lane ^ sh for sh in (8, 4, 2, 1)]

    def compute(j, b):
        off0, off = off_of(j)

        def grp_body(g, carry2):
            gbase = g * L
            gidx = off + gbase + lane
            segv = sgb[b][pl.ds(gbase, L)]
            seg2 = jnp.where(gidx >= off0, segv, B)
            s2b[b][pl.ds(gbase, L)] = seg2
            smv = smb[b][pl.ds(gbase, L)]
            wacc = jnp.zeros((L,), jnp.float32)
            for r in range(L):
                row = gbase + r
                chunks = [fb[b][row, pl.ds(kk * L, L)] for kk in range(D // L)]
                accv = bvn16 + chunks[0] * wv[0]
                for kk in range(1, D // L):
                    accv = accv + chunks[kk] * wv[kk]
                for idx in bfly:
                    accv = accv + jnp.take(accv, idx)
                sig = 1.0 / (1.0 + jnp.exp(accv))
                wrow = sig * smv[r]
                wacc = wacc + wrow * ohf[r]
                for kk in range(D // L):
                    fb[b][row, pl.ds(kk * L, L)] = chunks[kk] * wrow
            wob[b][pl.ds(gbase, L)] = wacc
            return carry2

        lax.fori_loop(0, R_BLK // L, grp_body, 0)

    def phase(j, b):
        @pl.when(j >= NBUF)
        def _():
            pltpu.make_async_copy(wob[b], wout.at[pl.ds(0, R_BLK)],
                                  sem_w.at[b]).wait()
        wait_in(j, b)
        b2 = (b + 2) % NBUF

        @pl.when(j + 2 < nblk)
        def _():
            @pl.when(j >= 2)
            def _():
                pltpu.make_async_copy(fb[b2], acc.at[s2b[b2]],
                                      sem_sc.at[b2]).wait()
            issue_in(j + 2, b2)

        compute(j, b)
        _, off = off_of(j)
        pltpu.async_copy(fb[b], acc.at[s2b[b]], sem_sc.at[b], add=True)
        pltpu.async_copy(wob[b], wout.at[pl.ds(off, R_BLK)], sem_w.at[b])

    nmac = (nblk + NBUF - 1) // NBUF

    def mac_body(m, carry):
        for p in range(NBUF):
            j = m * NBUF + p

            @pl.when(j < nblk)
            def _():
                phase(j, p)
        return carry

    lax.fori_loop(0, nmac, mac_body, 0)

    for b in range(NBUF):
        pltpu.make_async_copy(fb[b], acc.at[s2b[b]], sem_sc.at[b]).wait()
        pltpu.make_async_copy(wob[b], wout.at[pl.ds(0, R_BLK)],
                              sem_w.at[b]).wait()

    plsc.subcore_barrier()
    rpt = B // NS
    pltpu.sync_copy(acc.at[pl.ds(s * rpt, rpt)], part.at[c, pl.ds(s * rpt, rpt)])


_sc_call = pl.kernel(
    _sc_body,
    out_type=(
        jax.ShapeDtypeStruct((N,), jnp.float32),
        jax.ShapeDtypeStruct((NC, B, D), jnp.float32),
    ),
    mesh=plsc.VectorSubcoreMesh(
        core_axis_name="c", subcore_axis_name="s", num_cores=NC, num_subcores=NS
    ),
    scratch_types=(
        [pltpu.VMEM_SHARED((ACC_R, D), jnp.float32)]
        + [pltpu.VMEM((R_BLK, D), jnp.float32)] * NBUF
        + [pltpu.VMEM((R_BLK,), jnp.int32)] * NBUF
        + [pltpu.VMEM((R_BLK,), jnp.int32)] * NBUF
        + [pltpu.VMEM((R_BLK,), jnp.float32)] * NBUF
        + [pltpu.VMEM((R_BLK,), jnp.float32)] * NBUF
        + [pltpu.VMEM((D,), jnp.float32),
           pltpu.VMEM((L,), jnp.float32),
           pltpu.SemaphoreType.DMA((NBUF,)),
           pltpu.SemaphoreType.DMA((NBUF,)),
           pltpu.SemaphoreType.DMA((NBUF,))]
    ),
)


def _ffn_body(part, W1, b1, g1, be1, W2, b2, g2, be2, W3, b3, g3, be3,
              Wp, bp, out):
    h = part[0] + part[1]

    def fc(x, W, b, g, be):
        y = lax.dot_general(x, W[...], (((1,), (1,)), ((), ())),
                            preferred_element_type=jnp.float32) + b[...]
        y = jnp.maximum(y, 0.0)
        m = jnp.mean(y, axis=0, keepdims=True)
        v = jnp.mean(y * y, axis=0, keepdims=True) - m * m
        return (y - m) * lax.rsqrt(v + EPS) * g[...] + be[...]

    h = fc(h, W1, b1, g1, be1)
    h = fc(h, W2, b2, g2, be2)
    h = fc(h, W3, b3, g3, be3)
    out[...] = jnp.sum(h * Wp[...], axis=1, keepdims=True) + bp[...]


def _ffn_call(part, W1, b1, g1, be1, W2, b2, g2, be2, W3, b3, g3, be3, Wp, bp):
    return pl.pallas_call(
        _ffn_body,
        out_shape=jax.ShapeDtypeStruct((B, 1), jnp.float32),
    )(part, W1, b1, g1, be1, W2, b2, g2, be2, W3, b3, g3, be3, Wp, bp)


def kernel(segment_ids, rgcn_node_feats, rgcn_edge_feats, smask_feats,
           W_aw, b_aw, W1, b1, g1, be1, W2, b2, g2, be2, W3, b3, g3, be3,
           Wp, bp):
    seg = segment_ids.astype(jnp.int32)
    sm = smask_feats.reshape(N).astype(jnp.float32)
    wawn = (-W_aw).reshape(D).astype(jnp.float32)
    bawn = jnp.broadcast_to((-b_aw / L).astype(jnp.float32).reshape(1), (L,))

    wout, part = _sc_call(rgcn_node_feats, seg, sm, wawn, bawn)

    out = _ffn_call(part,
                    W1, b1.reshape(1, H), g1.reshape(1, H), be1.reshape(1, H),
                    W2, b2.reshape(1, H), g2.reshape(1, H), be2.reshape(1, H),
                    W3, b3.reshape(1, H), g3.reshape(1, H), be3.reshape(1, H),
                    Wp, bp.reshape(1, 1))
    return (out, wout.reshape(N, 1))
